# Initial kernel scaffold; baseline (speedup 1.0000x reference)
import functools

import jax
import jax.numpy as jnp
from jax import lax
from jax.experimental import pallas as pl
from jax.experimental.pallas import tpu as pltpu
from jax.experimental.pallas import tpu_sc as plsc

ROWS = 128
COLS = 32768
KTOP = 64
NC = 2
NS = 16
L = 16
NW = NC * NS
RPW = ROWS // NW
NVEC = COLS // L
NB1 = 4096
NB3 = 256

_MESH = plsc.VectorSubcoreMesh(
    core_axis_name="c", subcore_axis_name="s", num_cores=NC, num_subcores=NS
)

_MASK31 = jnp.int32(0x7FFFFFFF)


def _xlane(v, perm):
    return v.at[perm].get(mode="promise_in_bounds")


def _clean16(v, iota):
    for s in (8, 4, 2, 1):
        p = _xlane(v, iota ^ s)
        take_min = (iota & s) == 0
        v = jnp.where(take_min, jnp.minimum(v, p), jnp.maximum(v, p))
    return v


def _merge16(a, b, iota):
    br = lax.rev(b, (0,))
    lo = jnp.minimum(a, br)
    hi = jnp.maximum(a, br)
    return _clean16(lo, iota), _clean16(hi, iota)


def _sort64(d0, d1, d2, d3, iota):
    s0 = jnp.sort(d0)
    s1 = jnp.sort(d1)
    s2 = jnp.sort(d2)
    s3 = jnp.sort(d3)
    a0, a1 = _merge16(s0, s1, iota)
    b0, b1 = _merge16(s2, s3, iota)
    rb1 = lax.rev(b1, (0,))
    rb0 = lax.rev(b0, (0,))
    l0 = jnp.minimum(a0, rb1)
    l1 = jnp.minimum(a1, rb0)
    h0 = jnp.maximum(a0, rb1)
    h1 = jnp.maximum(a1, rb0)
    e0 = _clean16(jnp.minimum(l0, l1), iota)
    e1 = _clean16(jnp.maximum(l0, l1), iota)
    e2 = _clean16(jnp.minimum(h0, h1), iota)
    e3 = _clean16(jnp.maximum(h0, h1), iota)
    return e0, e1, e2, e3


def _body(x_hbm, out_hbm, x_v, keys_v, hist_v, def_v, out_v):
    wid = lax.axis_index("s") * NC + lax.axis_index("c")
    iota = lax.iota(jnp.int32, L)
    zeros16 = jnp.zeros((L,), jnp.int32)
    ones16 = jnp.ones((L,), jnp.int32)

    def zero_hist(nbuckets):
        def zr(i, c):
            hist_v[pl.ds(i * L, L)] = zeros16
            return c
        lax.fori_loop(0, nbuckets // L, zr, 0)

    def find_threshold(nbuckets, k):
        def blk(t, carry):
            running, fb, above = carry
            i = nbuckets // L - 1 - t
            bs = jnp.sum(hist_v[pl.ds(i * L, L)])
            hit = jnp.logical_and(fb < 0, running + bs >= k)
            return (
                running + bs,
                jnp.where(hit, i, fb),
                jnp.where(hit, running, above),
            )
        _, fb, above_blk = lax.fori_loop(
            0, nbuckets // L, blk,
            (jnp.int32(0), jnp.int32(-1), jnp.int32(0)),
        )

        def lane(t, carry):
            running, bst, above = carry
            b = fb * L + (L - 1 - t)
            hv = hist_v[b]
            hit = jnp.logical_and(bst < 0, running + hv >= k)
            return (
                running + hv,
                jnp.where(hit, b, bst),
                jnp.where(hit, running, above),
            )
        _, bst, above = lax.fori_loop(
            0, L, lane, (above_blk, jnp.int32(-1), jnp.int32(0))
        )
        return bst, above

    def refine_level(shift, bmask, nbuckets, count, k, cur_d):
        zero_hist(nbuckets)
        nv = (count + (L - 1)) // L

        def hst(i, c):
            sk = keys_v[pl.ds(i * L, L)]
            valid = (i * L + iota) < count
            b = lax.shift_right_arithmetic(sk, shift) & jnp.int32(bmask)
            plsc.addupdate_scatter(hist_v, [b], ones16, mask=valid)
            return c
        lax.fori_loop(0, nv, hst, 0)

        bst, above = find_threshold(nbuckets, k)

        def flt(i, carry):
            cd, cc = carry
            sk = keys_v[pl.ds(i * L, L)]
            valid = (i * L + iota) < count
            b = lax.shift_right_arithmetic(sk, shift) & jnp.int32(bmask)
            gt = jnp.logical_and(valid, b > bst)
            eq = jnp.logical_and(valid, b == bst)
            idx_d = cd + plsc.cumsum(gt.astype(jnp.int32)) - 1
            plsc.store_scatter(def_v, [idx_d], sk, mask=gt)
            idx_c = cc + plsc.cumsum(eq.astype(jnp.int32)) - 1
            plsc.store_scatter(keys_v, [idx_c], sk, mask=eq)
            return (
                cd + plsc.all_reduce_population_count(gt),
                cc + plsc.all_reduce_population_count(eq),
            )
        cur_d, cur_c = lax.fori_loop(0, nv, flt, (cur_d, zeros16))
        return cur_d, jnp.max(cur_c), bst, above

    def do_row(j, carry):
        r = wid * RPW + j
        pltpu.sync_copy(x_hbm.at[r], x_v)

        zero_hist(NB1)

        def p1(i, c):
            x = x_v[pl.ds(i * L, L)]
            u = plsc.bitcast(x, jnp.int32)
            sk = u ^ (lax.shift_right_arithmetic(u, 31) & _MASK31)
            keys_v[pl.ds(i * L, L)] = sk
            b = lax.shift_right_arithmetic(sk, 20) + 2048
            plsc.addupdate_scatter(hist_v, [b], ones16)
            return c
        lax.fori_loop(0, NVEC, p1, 0)

        bs1, above1 = find_threshold(NB1, jnp.int32(KTOP))

        def f1(i, carry):
            cd, cc = carry
            sk = keys_v[pl.ds(i * L, L)]
            b = lax.shift_right_arithmetic(sk, 20) + 2048
            hot = b >= bs1

            def slow(c):
                cd2, cc2 = c
                gt = b > bs1
                eq = b == bs1
                idx_d = cd2 + plsc.cumsum(gt.astype(jnp.int32)) - 1
                plsc.store_scatter(def_v, [idx_d], sk, mask=gt)
                idx_c = cc2 + plsc.cumsum(eq.astype(jnp.int32)) - 1
                plsc.store_scatter(keys_v, [idx_c], sk, mask=eq)
                return (
                    cd2 + plsc.all_reduce_population_count(gt),
                    cc2 + plsc.all_reduce_population_count(eq),
                )
            return lax.cond(jnp.any(hot), slow, lambda c: c, (cd, cc))
        cur_d, cur_c = lax.fori_loop(0, NVEC, f1, (zeros16, zeros16))
        c1 = jnp.max(cur_c)
        k1 = jnp.int32(KTOP) - above1

        cur_d, c2, bs2, above2 = refine_level(8, 0xFFF, NB1, c1, k1, cur_d)
        k2 = k1 - above2
        cur_d, _c3, bs3, above3 = refine_level(0, 0xFF, NB3, c2, k2, cur_d)
        k3 = k2 - above3

        t_key = (
            lax.shift_left(bs1 - 2048, 20)
            | lax.shift_left(bs2, 8)
            | bs3
        )
        for t in range(4):
            m = (t * L + iota) < k3
            idx = cur_d + t * L + iota
            plsc.store_scatter(def_v, [idx], jnp.full((L,), t_key), mask=m)

        d0 = def_v[pl.ds(0, L)]
        d1 = def_v[pl.ds(L, L)]
        d2 = def_v[pl.ds(2 * L, L)]
        d3 = def_v[pl.ds(3 * L, L)]
        e0, e1, e2, e3 = _sort64(d0, d1, d2, d3, iota)
        for t, e in enumerate((e3, e2, e1, e0)):
            w = lax.rev(e, (0,))
            u = w ^ (lax.shift_right_arithmetic(w, 31) & _MASK31)
            out_v[pl.ds(t * L, L)] = plsc.bitcast(u, jnp.float32)
        pltpu.sync_copy(out_v, out_hbm.at[r])
        return carry

    lax.fori_loop(0, RPW, do_row, 0)


_topk_sc = functools.partial(
    pl.kernel,
    out_type=jax.ShapeDtypeStruct((ROWS, KTOP), jnp.float32),
    mesh=_MESH,
    scratch_types=[
        pltpu.VMEM((COLS,), jnp.float32),
        pltpu.VMEM((COLS,), jnp.int32),
        pltpu.VMEM((NB1,), jnp.int32),
        pltpu.VMEM((2 * KTOP,), jnp.int32),
        pltpu.VMEM((KTOP,), jnp.float32),
    ],
)(_body)


def kernel(input):
    return _topk_sc(input)


# speedup vs baseline: 1.6349x; 1.6349x over previous
import functools

import jax
import jax.numpy as jnp
import numpy as np
from jax import lax
from jax.experimental import pallas as pl
from jax.experimental.pallas import tpu as pltpu
from jax.experimental.pallas import tpu_sc as plsc

ROWS = 128
COLS = 32768
KTOP = 64
NC = 2
NS = 16
L = 16
NW = NC * NS
RPW = ROWS // NW
NVEC = COLS // L
NB1 = 4096
NB3 = 256

_MESH = plsc.VectorSubcoreMesh(
    core_axis_name="c", subcore_axis_name="s", num_cores=NC, num_subcores=NS
)

_MASK31 = np.int32(0x7FFFFFFF)


def _xlane(v, perm):
    return v.at[perm].get(mode="promise_in_bounds")


def _clean16(v, iota):
    for s in (8, 4, 2, 1):
        p = _xlane(v, iota ^ s)
        take_min = (iota & s) == 0
        v = jnp.where(take_min, jnp.minimum(v, p), jnp.maximum(v, p))
    return v


def _merge16(a, b, iota):
    br = lax.rev(b, (0,))
    lo = jnp.minimum(a, br)
    hi = jnp.maximum(a, br)
    return _clean16(lo, iota), _clean16(hi, iota)


def _sort64(d0, d1, d2, d3, iota):
    s0 = jnp.sort(d0)
    s1 = jnp.sort(d1)
    s2 = jnp.sort(d2)
    s3 = jnp.sort(d3)
    a0, a1 = _merge16(s0, s1, iota)
    b0, b1 = _merge16(s2, s3, iota)
    rb1 = lax.rev(b1, (0,))
    rb0 = lax.rev(b0, (0,))
    l0 = jnp.minimum(a0, rb1)
    l1 = jnp.minimum(a1, rb0)
    h0 = jnp.maximum(a0, rb1)
    h1 = jnp.maximum(a1, rb0)
    e0 = _clean16(jnp.minimum(l0, l1), iota)
    e1 = _clean16(jnp.maximum(l0, l1), iota)
    e2 = _clean16(jnp.minimum(h0, h1), iota)
    e3 = _clean16(jnp.maximum(h0, h1), iota)
    return e0, e1, e2, e3


def _body(x_hbm, out_hbm, x_v, keys_v, hist_v, def_v, out_v):
    wid = lax.axis_index("s") * NC + lax.axis_index("c")
    iota = lax.iota(jnp.int32, L)
    zeros16 = jnp.zeros((L,), jnp.int32)
    ones16 = jnp.ones((L,), jnp.int32)

    def zero_hist(nbuckets):
        def zr(i, c):
            hist_v[pl.ds(i * L, L)] = zeros16
            return c
        lax.fori_loop(0, nbuckets // L, zr, 0)

    def find_threshold(nbuckets, k):
        def blk(t, carry):
            running, fb, above = carry
            i = nbuckets // L - 1 - t
            bs = jnp.sum(hist_v[pl.ds(i * L, L)])
            hit = jnp.logical_and(fb < 0, running + bs >= k)
            return (
                running + bs,
                jnp.where(hit, i, fb),
                jnp.where(hit, running, above),
            )
        _, fb, above_blk = lax.fori_loop(
            0, nbuckets // L, blk,
            (jnp.int32(0), jnp.int32(-1), jnp.int32(0)),
        )
        h = hist_v[pl.ds(fb * L, L)]
        hr = lax.rev(h, (0,))
        c = plsc.cumsum(hr)
        crossed = (above_blk + c) >= k
        ts = plsc.all_reduce_ffs(crossed)
        bst = fb * L + (L - 1) - ts[0]
        cs = _xlane(c, ts)
        hs = _xlane(hr, ts)
        above = above_blk + cs[0] - hs[0]
        return bst, above

    def refine_level(shift, bmask, nbuckets, count, k, cur_d):
        zero_hist(nbuckets)
        nv = (count + (L - 1)) // L

        def hst(i, c):
            sk = keys_v[pl.ds(i * L, L)]
            valid = (i * L + iota) < count
            b = lax.shift_right_arithmetic(sk, shift) & jnp.int32(bmask)
            plsc.addupdate_scatter(hist_v, [b], ones16, mask=valid)
            return c
        lax.fori_loop(0, nv, hst, 0)

        bst, above = find_threshold(nbuckets, k)

        def flt(i, carry):
            cd, cc = carry
            sk = keys_v[pl.ds(i * L, L)]
            valid = (i * L + iota) < count
            b = lax.shift_right_arithmetic(sk, shift) & jnp.int32(bmask)
            gt = jnp.logical_and(valid, b > bst)
            eq = jnp.logical_and(valid, b == bst)
            idx_d = cd + plsc.cumsum(gt.astype(jnp.int32)) - 1
            plsc.store_scatter(def_v, [idx_d], sk, mask=gt)
            idx_c = cc + plsc.cumsum(eq.astype(jnp.int32)) - 1
            plsc.store_scatter(keys_v, [idx_c], sk, mask=eq)
            return (
                cd + plsc.all_reduce_population_count(gt),
                cc + plsc.all_reduce_population_count(eq),
            )
        cur_d, cur_c = lax.fori_loop(0, nv, flt, (cur_d, zeros16))
        return cur_d, jnp.max(cur_c), bst, above

    def do_row(j, carry):
        r = wid * RPW + j
        pltpu.sync_copy(x_hbm.at[r], x_v)

        zero_hist(NB1)

        def p1(i, c):
            x = x_v[pl.ds(i * L, L)]
            u = lax.bitcast_convert_type(x, jnp.int32)
            sk = u ^ (lax.shift_right_arithmetic(u, 31) & _MASK31)
            keys_v[pl.ds(i * L, L)] = sk
            b = lax.shift_right_arithmetic(sk, 20) + 2048
            plsc.addupdate_scatter(hist_v, [b], ones16)
            return c
        lax.fori_loop(0, NVEC, p1, 0)

        bs1, above1 = find_threshold(NB1, jnp.int32(KTOP))

        def f1(i, carry):
            cd, cc = carry
            sk = keys_v[pl.ds(i * L, L)]
            b = lax.shift_right_arithmetic(sk, 20) + 2048
            hot = b >= bs1

            def slow(c):
                cd2, cc2 = c
                gt = b > bs1
                eq = b == bs1
                idx_d = cd2 + plsc.cumsum(gt.astype(jnp.int32)) - 1
                plsc.store_scatter(def_v, [idx_d], sk, mask=gt)
                idx_c = cc2 + plsc.cumsum(eq.astype(jnp.int32)) - 1
                plsc.store_scatter(keys_v, [idx_c], sk, mask=eq)
                return (
                    cd2 + plsc.all_reduce_population_count(gt),
                    cc2 + plsc.all_reduce_population_count(eq),
                )
            return lax.cond(jnp.any(hot), slow, lambda c: c, (cd, cc))
        cur_d, cur_c = lax.fori_loop(0, NVEC, f1, (zeros16, zeros16))
        c1 = jnp.max(cur_c)
        k1 = jnp.int32(KTOP) - above1

        cur_d, c2, bs2, above2 = refine_level(8, 0xFFF, NB1, c1, k1, cur_d)
        k2 = k1 - above2
        cur_d, _c3, bs3, above3 = refine_level(0, 0xFF, NB3, c2, k2, cur_d)
        k3 = k2 - above3

        t_key = (
            lax.shift_left(bs1 - 2048, 20)
            | lax.shift_left(bs2, 8)
            | bs3
        )
        for t in range(4):
            m = (t * L + iota) < k3
            idx = cur_d + t * L + iota
            plsc.store_scatter(def_v, [idx], jnp.full((L,), t_key), mask=m)

        d0 = def_v[pl.ds(0, L)]
        d1 = def_v[pl.ds(L, L)]
        d2 = def_v[pl.ds(2 * L, L)]
        d3 = def_v[pl.ds(3 * L, L)]
        e0, e1, e2, e3 = _sort64(d0, d1, d2, d3, iota)
        for t, e in enumerate((e3, e2, e1, e0)):
            w = lax.rev(e, (0,))
            u = w ^ (lax.shift_right_arithmetic(w, 31) & _MASK31)
            out_v[pl.ds(t * L, L)] = lax.bitcast_convert_type(u, jnp.float32)
        pltpu.sync_copy(out_v, out_hbm.at[r])
        return carry

    lax.fori_loop(0, RPW, do_row, 0)


_topk_sc = functools.partial(
    pl.kernel,
    out_type=jax.ShapeDtypeStruct((ROWS, KTOP), jnp.float32),
    mesh=_MESH,
    compiler_params=pltpu.CompilerParams(needs_layout_passes=False),
    scratch_types=[
        pltpu.VMEM((COLS,), jnp.float32),
        pltpu.VMEM((COLS,), jnp.int32),
        pltpu.VMEM((NB1,), jnp.int32),
        pltpu.VMEM((2 * KTOP,), jnp.int32),
        pltpu.VMEM((KTOP,), jnp.float32),
    ],
)(_body)


def kernel(input):
    return _topk_sc(input)


# speedup vs baseline: 3.5940x; 2.1983x over previous
import functools

import jax
import jax.numpy as jnp
import numpy as np
from jax import lax
from jax.experimental import pallas as pl
from jax.experimental.pallas import tpu as pltpu
from jax.experimental.pallas import tpu_sc as plsc

ROWS = 128
COLS = 32768
KTOP = 64
NC = 2
NS = 16
L = 16
NW = NC * NS
RPW = ROWS // NW
NVEC = COLS // L
NB1 = 4096
NB3 = 256
SSTRIDE = 16
SMIN = 16

_MESH = plsc.VectorSubcoreMesh(
    core_axis_name="c", subcore_axis_name="s", num_cores=NC, num_subcores=NS
)

_MASK31 = np.int32(0x7FFFFFFF)


def _keyize(u):
    return u ^ (lax.shift_right_arithmetic(u, 31) & _MASK31)


def _xlane(v, perm):
    return v.at[perm].get(mode="promise_in_bounds")


def _clean16(v, iota):
    for s in (8, 4, 2, 1):
        p = _xlane(v, iota ^ s)
        take_min = (iota & s) == 0
        v = jnp.where(take_min, jnp.minimum(v, p), jnp.maximum(v, p))
    return v


def _merge16(a, b, iota):
    br = lax.rev(b, (0,))
    lo = jnp.minimum(a, br)
    hi = jnp.maximum(a, br)
    return _clean16(lo, iota), _clean16(hi, iota)


def _sort64(d0, d1, d2, d3, iota):
    s0 = jnp.sort(d0)
    s1 = jnp.sort(d1)
    s2 = jnp.sort(d2)
    s3 = jnp.sort(d3)
    a0, a1 = _merge16(s0, s1, iota)
    b0, b1 = _merge16(s2, s3, iota)
    rb1 = lax.rev(b1, (0,))
    rb0 = lax.rev(b0, (0,))
    l0 = jnp.minimum(a0, rb1)
    l1 = jnp.minimum(a1, rb0)
    h0 = jnp.maximum(a0, rb1)
    h1 = jnp.maximum(a1, rb0)
    e0 = _clean16(jnp.minimum(l0, l1), iota)
    e1 = _clean16(jnp.maximum(l0, l1), iota)
    e2 = _clean16(jnp.minimum(h0, h1), iota)
    e3 = _clean16(jnp.maximum(h0, h1), iota)
    return e0, e1, e2, e3


def _body(x_hbm, out_hbm, x_v, cand_v, hist_v, def_v, out_v):
    wid = lax.axis_index("s") * NC + lax.axis_index("c")
    iota = lax.iota(jnp.int32, L)
    zeros16 = jnp.zeros((L,), jnp.int32)
    ones16 = jnp.ones((L,), jnp.int32)

    def zero_hist(nbuckets):
        def zr(i, c):
            hist_v[pl.ds(i * L, L)] = zeros16
            return c
        lax.fori_loop(0, nbuckets // L, zr, 0)

    def find_threshold(nbuckets, k):
        def blk(t, carry):
            running, fb, above = carry
            i = nbuckets // L - 1 - t
            bs = jnp.sum(hist_v[pl.ds(i * L, L)])
            hit = jnp.logical_and(fb < 0, running + bs >= k)
            return (
                running + bs,
                jnp.where(hit, i, fb),
                jnp.where(hit, running, above),
            )
        _, fb, above_blk = lax.fori_loop(
            0, nbuckets // L, blk,
            (jnp.int32(0), jnp.int32(-1), jnp.int32(0)),
        )
        h = hist_v[pl.ds(fb * L, L)]
        hr = lax.rev(h, (0,))
        c = plsc.cumsum(hr)
        crossed = (above_blk + c) >= k
        ts = plsc.all_reduce_ffs(crossed)
        bst = fb * L + (L - 1) - ts[0]
        cs = _xlane(c, ts)
        hs = _xlane(hr, ts)
        above = above_blk + cs[0] - hs[0]
        return bst, above

    def refine_level(bucket_fn, nbuckets, count, k, cur_d):
        zero_hist(nbuckets)
        nv = (count + (L - 1)) // L

        def hst(i, c):
            sk = cand_v[pl.ds(i * L, L)]
            valid = (i * L + iota) < count
            plsc.addupdate_scatter(hist_v, [bucket_fn(sk)], ones16, mask=valid)
            return c
        lax.fori_loop(0, nv, hst, 0)

        bst, above = find_threshold(nbuckets, k)

        def flt(i, carry):
            cd, cc = carry
            sk = cand_v[pl.ds(i * L, L)]
            valid = (i * L + iota) < count
            b = bucket_fn(sk)
            gt = jnp.logical_and(valid, b > bst)
            eq = jnp.logical_and(valid, b == bst)
            idx_d = cd + plsc.cumsum(gt.astype(jnp.int32)) - 1
            plsc.store_scatter(def_v, [idx_d], sk, mask=gt)
            idx_c = cc + plsc.cumsum(eq.astype(jnp.int32)) - 1
            plsc.store_scatter(cand_v, [idx_c], sk, mask=eq)
            return (
                cd + plsc.all_reduce_population_count(gt),
                cc + plsc.all_reduce_population_count(eq),
            )
        cur_d, cur_c = lax.fori_loop(0, nv, flt, (cur_d, zeros16))
        return cur_d, jnp.max(cur_c), bst, above

    def bucket_top12(sk):
        return lax.shift_right_arithmetic(sk, 20) + 2048

    def bucket_mid12(sk):
        return lax.shift_right_arithmetic(sk, 8) & jnp.int32(0xFFF)

    def bucket_low8(sk):
        return sk & jnp.int32(0xFF)

    def do_row(j, carry):
        r = wid * RPW + j
        pltpu.sync_copy(x_hbm.at[r], x_v)

        zero_hist(NB1)

        def samp(i, c):
            x = x_v[pl.ds(i * (SSTRIDE * L), L)]
            sk = _keyize(lax.bitcast_convert_type(x, jnp.int32))
            plsc.addupdate_scatter(hist_v, [bucket_top12(sk)], ones16)
            return c
        lax.fori_loop(0, NVEC // SSTRIDE, samp, 0)
        bst_s, _ = find_threshold(NB1, jnp.int32(SMIN))
        tk = lax.shift_left(bst_s - 2048, 20)
        t_low = lax.bitcast_convert_type(tk ^ (lax.shift_right_arithmetic(tk, 31) & _MASK31), jnp.float32)

        def compact_pass(t):
            def pb(i, cc):
                for u in range(4):
                    x = x_v[pl.ds((4 * i + u) * L, L)]
                    hot = x >= t
                    idx = cc + plsc.cumsum(hot.astype(jnp.int32)) - 1
                    plsc.store_scatter(
                        cand_v, [idx],
                        lax.bitcast_convert_type(x, jnp.int32), mask=hot)
                    cc = cc + plsc.all_reduce_population_count(hot)
                return cc
            return jnp.max(lax.fori_loop(0, NVEC // 4, pb, zeros16))

        count = compact_pass(t_low)
        count = lax.cond(
            count < KTOP,
            lambda: compact_pass(jnp.float32(-jnp.inf)),
            lambda: count,
        )

        def kz(i, c):
            u = cand_v[pl.ds(i * L, L)]
            cand_v[pl.ds(i * L, L)] = _keyize(u)
            return c
        lax.fori_loop(0, (count + (L - 1)) // L, kz, 0)

        cur_d, c1, bs1, above1 = refine_level(
            bucket_top12, NB1, count, jnp.int32(KTOP), zeros16)
        k1 = jnp.int32(KTOP) - above1
        cur_d, c2, bs2, above2 = refine_level(bucket_mid12, NB1, c1, k1, cur_d)
        k2 = k1 - above2
        cur_d, _c3, bs3, above3 = refine_level(bucket_low8, NB3, c2, k2, cur_d)
        k3 = k2 - above3

        t_key = (
            lax.shift_left(bs1 - 2048, 20)
            | lax.shift_left(bs2, 8)
            | bs3
        )
        for t in range(4):
            m = (t * L + iota) < k3
            idx = cur_d + t * L + iota
            plsc.store_scatter(def_v, [idx], jnp.full((L,), t_key), mask=m)

        d0 = def_v[pl.ds(0, L)]
        d1 = def_v[pl.ds(L, L)]
        d2 = def_v[pl.ds(2 * L, L)]
        d3 = def_v[pl.ds(3 * L, L)]
        e0, e1, e2, e3 = _sort64(d0, d1, d2, d3, iota)
        for t, e in enumerate((e3, e2, e1, e0)):
            w = lax.rev(e, (0,))
            u = w ^ (lax.shift_right_arithmetic(w, 31) & _MASK31)
            out_v[pl.ds(t * L, L)] = lax.bitcast_convert_type(u, jnp.float32)
        pltpu.sync_copy(out_v, out_hbm.at[r])
        return carry

    lax.fori_loop(0, RPW, do_row, 0)


_topk_sc = functools.partial(
    pl.kernel,
    out_type=jax.ShapeDtypeStruct((ROWS, KTOP), jnp.float32),
    mesh=_MESH,
    compiler_params=pltpu.CompilerParams(needs_layout_passes=False),
    scratch_types=[
        pltpu.VMEM((COLS,), jnp.float32),
        pltpu.VMEM((COLS,), jnp.int32),
        pltpu.VMEM((NB1,), jnp.int32),
        pltpu.VMEM((2 * KTOP,), jnp.int32),
        pltpu.VMEM((KTOP,), jnp.float32),
    ],
)(_body)


def kernel(input):
    return _topk_sc(input)


# speedup vs baseline: 3.9741x; 1.1058x over previous
import functools

import jax
import jax.numpy as jnp
import numpy as np
from jax import lax
from jax.experimental import pallas as pl
from jax.experimental.pallas import tpu as pltpu
from jax.experimental.pallas import tpu_sc as plsc

ROWS = 128
COLS = 32768
KTOP = 64
NC = 2
NS = 16
L = 16
NW = NC * NS
RPW = ROWS // NW
NVEC = COLS // L
NB = 256
SSTRIDE = 16
SMIN = 16

_MESH = plsc.VectorSubcoreMesh(
    core_axis_name="c", subcore_axis_name="s", num_cores=NC, num_subcores=NS
)

_MASK31 = np.int32(0x7FFFFFFF)


def _keyize(u):
    return u ^ (lax.shift_right_arithmetic(u, 31) & _MASK31)


def _xlane(v, perm):
    return v.at[perm].get(mode="promise_in_bounds")


def _clean16(v, iota):
    for s in (8, 4, 2, 1):
        p = _xlane(v, iota ^ s)
        take_min = (iota & s) == 0
        v = jnp.where(take_min, jnp.minimum(v, p), jnp.maximum(v, p))
    return v


def _merge16(a, b, iota):
    br = lax.rev(b, (0,))
    lo = jnp.minimum(a, br)
    hi = jnp.maximum(a, br)
    return _clean16(lo, iota), _clean16(hi, iota)


def _sort64(d0, d1, d2, d3, iota):
    s0 = jnp.sort(d0)
    s1 = jnp.sort(d1)
    s2 = jnp.sort(d2)
    s3 = jnp.sort(d3)
    a0, a1 = _merge16(s0, s1, iota)
    b0, b1 = _merge16(s2, s3, iota)
    rb1 = lax.rev(b1, (0,))
    rb0 = lax.rev(b0, (0,))
    l0 = jnp.minimum(a0, rb1)
    l1 = jnp.minimum(a1, rb0)
    h0 = jnp.maximum(a0, rb1)
    h1 = jnp.maximum(a1, rb0)
    e0 = _clean16(jnp.minimum(l0, l1), iota)
    e1 = _clean16(jnp.maximum(l0, l1), iota)
    e2 = _clean16(jnp.minimum(h0, h1), iota)
    e3 = _clean16(jnp.maximum(h0, h1), iota)
    return e0, e1, e2, e3


def _body(x_hbm, out_hbm, xa_v, xb_v, cand_v, hist_v, def_v, out_v, sa, sb):
    wid = lax.axis_index("s") * NC + lax.axis_index("c")
    iota = lax.iota(jnp.int32, L)
    zeros16 = jnp.zeros((L,), jnp.int32)
    ones16 = jnp.ones((L,), jnp.int32)

    def zero_hist():
        for i in range(NB // L):
            hist_v[pl.ds(i * L, L)] = zeros16

    def find_threshold(k):
        def blk(t, carry):
            running, fb, above = carry
            i = NB // L - 1 - t
            bs = jnp.sum(hist_v[pl.ds(i * L, L)])
            hit = jnp.logical_and(fb < 0, running + bs >= k)
            return (
                running + bs,
                jnp.where(hit, i, fb),
                jnp.where(hit, running, above),
            )
        _, fb, above_blk = lax.fori_loop(
            0, NB // L, blk, (jnp.int32(0), jnp.int32(-1), jnp.int32(0))
        )
        h = hist_v[pl.ds(fb * L, L)]
        hr = lax.rev(h, (0,))
        c = plsc.cumsum(hr)
        crossed = (above_blk + c) >= k
        ts = plsc.all_reduce_ffs(crossed)
        bst = fb * L + (L - 1) - ts[0]
        cs = _xlane(c, ts)
        hs = _xlane(hr, ts)
        above = above_blk + cs[0] - hs[0]
        return bst, above

    def refine_level(bucket_fn, count, k, cur_d):
        zero_hist()
        nv = (count + (L - 1)) // L

        def hst(i, c):
            sk = cand_v[pl.ds(i * L, L)]
            valid = (i * L + iota) < count
            plsc.addupdate_scatter(hist_v, [bucket_fn(sk)], ones16, mask=valid)
            return c
        lax.fori_loop(0, nv, hst, 0)

        bst, above = find_threshold(k)

        def flt(i, carry):
            cd, cc = carry
            sk = cand_v[pl.ds(i * L, L)]
            valid = (i * L + iota) < count
            b = bucket_fn(sk)
            gt = jnp.logical_and(valid, b > bst)
            eq = jnp.logical_and(valid, b == bst)
            idx_d = cd + plsc.cumsum(gt.astype(jnp.int32)) - 1
            plsc.store_scatter(def_v, [idx_d], sk, mask=gt)
            idx_c = cc + plsc.cumsum(eq.astype(jnp.int32)) - 1
            plsc.store_scatter(cand_v, [idx_c], sk, mask=eq)
            return (
                cd + plsc.all_reduce_population_count(gt),
                cc + plsc.all_reduce_population_count(eq),
            )
        cur_d, cur_c = lax.fori_loop(0, nv, flt, (cur_d, zeros16))
        return cur_d, jnp.max(cur_c), bst, above

    def bucket_b1(sk):
        return lax.shift_right_arithmetic(sk, 24) + 128

    def bucket_b2(sk):
        return lax.shift_right_arithmetic(sk, 16) & jnp.int32(0xFF)

    def bucket_b3(sk):
        return lax.shift_right_arithmetic(sk, 8) & jnp.int32(0xFF)

    def bucket_b4(sk):
        return sk & jnp.int32(0xFF)

    def process_row(x_v, r):
        zero_hist()

        def samp(i, c):
            x = x_v[pl.ds(i * (SSTRIDE * L), L)]
            sk = _keyize(lax.bitcast_convert_type(x, jnp.int32))
            plsc.addupdate_scatter(hist_v, [bucket_b1(sk)], ones16)
            return c
        lax.fori_loop(0, NVEC // SSTRIDE, samp, 0)
        bst_s, _ = find_threshold(jnp.int32(SMIN))
        tk = lax.shift_left(bst_s - 128, 24)
        t_low = lax.bitcast_convert_type(
            tk ^ (lax.shift_right_arithmetic(tk, 31) & _MASK31), jnp.float32)

        def compact_pass(t):
            def pb(i, cc):
                for u in range(4):
                    x = x_v[pl.ds((4 * i + u) * L, L)]
                    hot = x >= t
                    idx = cc + plsc.cumsum(hot.astype(jnp.int32)) - 1
                    plsc.store_scatter(
                        cand_v, [idx],
                        lax.bitcast_convert_type(x, jnp.int32), mask=hot)
                    cc = cc + plsc.all_reduce_population_count(hot)
                return cc
            return jnp.max(lax.fori_loop(0, NVEC // 4, pb, zeros16))

        count = compact_pass(t_low)
        count = lax.cond(
            count < KTOP,
            lambda: compact_pass(jnp.float32(-jnp.inf)),
            lambda: count,
        )

        def kz(i, c):
            u = cand_v[pl.ds(i * L, L)]
            cand_v[pl.ds(i * L, L)] = _keyize(u)
            return c
        lax.fori_loop(0, (count + (L - 1)) // L, kz, 0)

        cur_d, c1, bs1, above1 = refine_level(
            bucket_b1, count, jnp.int32(KTOP), zeros16)
        k1 = jnp.int32(KTOP) - above1
        cur_d, c2, bs2, above2 = refine_level(bucket_b2, c1, k1, cur_d)
        k2 = k1 - above2
        cur_d, c3, bs3, above3 = refine_level(bucket_b3, c2, k2, cur_d)
        k3 = k2 - above3
        cur_d, _c4, bs4, above4 = refine_level(bucket_b4, c3, k3, cur_d)
        k4 = k3 - above4

        t_key = (
            lax.shift_left(bs1 - 128, 24)
            | lax.shift_left(bs2, 16)
            | lax.shift_left(bs3, 8)
            | bs4
        )
        for t in range(4):
            m = (t * L + iota) < k4
            idx = cur_d + t * L + iota
            plsc.store_scatter(def_v, [idx], jnp.full((L,), t_key), mask=m)

        d0 = def_v[pl.ds(0, L)]
        d1 = def_v[pl.ds(L, L)]
        d2 = def_v[pl.ds(2 * L, L)]
        d3 = def_v[pl.ds(3 * L, L)]
        e0, e1, e2, e3 = _sort64(d0, d1, d2, d3, iota)
        for t, e in enumerate((e3, e2, e1, e0)):
            w = lax.rev(e, (0,))
            u = w ^ (lax.shift_right_arithmetic(w, 31) & _MASK31)
            out_v[pl.ds(t * L, L)] = lax.bitcast_convert_type(u, jnp.float32)
        pltpu.sync_copy(out_v, out_hbm.at[r])

    bufs = (xa_v, xb_v)
    sems = (sa, sb)
    r0 = wid * RPW
    cp = pltpu.async_copy(x_hbm.at[r0], bufs[0], sems[0])
    for j in range(RPW):
        cp.wait()
        if j + 1 < RPW:
            cp = pltpu.async_copy(
                x_hbm.at[r0 + j + 1], bufs[(j + 1) % 2], sems[(j + 1) % 2])
        process_row(bufs[j % 2], r0 + j)


_topk_sc = functools.partial(
    pl.kernel,
    out_type=jax.ShapeDtypeStruct((ROWS, KTOP), jnp.float32),
    mesh=_MESH,
    compiler_params=pltpu.CompilerParams(needs_layout_passes=False),
    scratch_types=[
        pltpu.VMEM((COLS,), jnp.float32),
        pltpu.VMEM((COLS,), jnp.float32),
        pltpu.VMEM((COLS,), jnp.int32),
        pltpu.VMEM((NB,), jnp.int32),
        pltpu.VMEM((2 * KTOP,), jnp.int32),
        pltpu.VMEM((KTOP,), jnp.float32),
        pltpu.SemaphoreType.DMA,
        pltpu.SemaphoreType.DMA,
    ],
)(_body)


def kernel(input):
    return _topk_sc(input)
